# baseline (speedup 1.0000x reference)
import functools

import jax
import jax.numpy as jnp
from jax import lax
from jax.experimental import pallas as pl
from jax.experimental.pallas import tpu as pltpu
from jax.experimental.pallas import tpu_sc as plsc

TOPK = 8


@functools.cache
def _make_sc_gather(V, D, B):
    info = plsc.get_sparse_core_info()
    NW = info.num_cores * info.num_subcores
    assert B % NW == 0
    b_per_w = B // NW
    CH = min(64, b_per_w)
    assert b_per_w % CH == 0
    n_ch = b_per_w // CH
    mesh = plsc.VectorSubcoreMesh(core_axis_name="c", subcore_axis_name="s")

    @functools.partial(
        pl.kernel,
        mesh=mesh,
        out_type=jax.ShapeDtypeStruct((B, D), jnp.float32),
        scratch_types=[
            pltpu.VMEM((b_per_w,), jnp.int32),
            pltpu.VMEM((2, CH, D), jnp.float32),
            pltpu.SemaphoreType.DMA((2,)),
            pltpu.SemaphoreType.DMA((2,)),
        ],
    )
    def gather_k(table_hbm, idx_hbm, out_hbm, idx_v, rows, gsem, osem):
        wid = lax.axis_index("s") * info.num_cores + lax.axis_index("c")
        base = wid * b_per_w
        pltpu.sync_copy(idx_hbm.at[pl.ds(base, b_per_w)], idx_v)

        def g_copy(c, slot):
            return pltpu.make_async_copy(
                table_hbm.at[idx_v.at[pl.ds(c * CH, CH)]],
                rows.at[slot],
                gsem.at[slot],
            )

        def o_copy(c, slot):
            return pltpu.make_async_copy(
                rows.at[slot],
                out_hbm.at[pl.ds(base + c * CH, CH)],
                osem.at[slot],
            )

        g_copy(0, 0).start()
        for c in range(n_ch):
            slot = c % 2
            if c + 1 < n_ch:
                if c >= 1:
                    o_copy(c - 1, 1 - slot).wait()
                g_copy(c + 1, 1 - slot).start()
            g_copy(c, slot).wait()
            o_copy(c, slot).start()
        if n_ch >= 2:
            o_copy(n_ch - 2, n_ch % 2).wait()
        o_copy(n_ch - 1, (n_ch - 1) % 2).wait()

    return gather_k


_SORT8 = (
    (0, 1), (2, 3), (4, 5), (6, 7),
    (0, 2), (1, 3), (4, 6), (5, 7),
    (1, 2), (5, 6),
    (0, 4), (1, 5), (2, 6), (3, 7),
    (2, 4), (3, 5),
    (1, 2), (3, 4), (5, 6),
)
_BITONIC8 = (
    (0, 4), (1, 5), (2, 6), (3, 7),
    (0, 2), (1, 3), (4, 6), (5, 7),
    (0, 1), (2, 3), (4, 5), (6, 7),
)


def _cmpswap(rows, net):
    for a, b in net:
        hi = jnp.maximum(rows[a], rows[b])
        lo = jnp.minimum(rows[a], rows[b])
        rows[a], rows[b] = hi, lo
    return rows


def _merge_top8(a, b):
    c = [jnp.maximum(a[i], b[7 - i]) for i in range(8)]
    return _cmpswap(c, _BITONIC8)


def _roll0(x, k):
    return jnp.concatenate([x[k:], x[:k]], axis=0)


def _top8_vals(xt):
    C, R = xt.shape
    nvr = C // 8
    rows = [xt[8 * v : 8 * v + 8] for v in range(nvr)]
    lists = []
    for g in range(nvr // 8):
        lists.append(_cmpswap(rows[8 * g : 8 * g + 8], _SORT8))
    while len(lists) > 1:
        lists = [
            _merge_top8(lists[i], lists[i + 1])
            for i in range(0, len(lists), 2)
        ]
    lst = lists[0]
    for d in (4, 2, 1):
        rolled = [_roll0(x, d) for x in lst]
        lst = _cmpswap(
            [jnp.maximum(lst[i], rolled[7 - i]) for i in range(8)], _BITONIC8
        )
    return lst[0], lst[7]


def _top8_softmax_weights_t(xt, tri):
    C, R = xt.shape
    v0, v8 = _top8_vals(xt)
    x3 = xt.reshape(C // 8, 8, R)
    sel = (x3 >= v8[None]).reshape(C, R)
    eqm = (x3 == v8[None]).reshape(C, R).astype(jnp.float32)
    cnt = jnp.sum(sel.astype(jnp.float32), axis=0, keepdims=True)
    need = cnt - TOPK
    r = lax.dot_general(
        tri[:C, :C], eqm, (((1,), (0,)), ((), ())),
        preferred_element_type=jnp.float32,
    )
    drop = jnp.logical_and(eqm > 0.0, r <= need)
    keep = jnp.logical_and(sel, jnp.logical_not(drop))
    e = jnp.where(keep, jnp.exp(x3 - v0[None]).reshape(C, R), 0.0)
    return e / jnp.sum(e, axis=0, keepdims=True)


_CONTRACT0 = (((0,), (0,)), ((), ()))


def _tc_body(g_ref, cm2_ref, root_ref, out_ref, l1_ref, tri_ref):
    E = out_ref.shape[1]

    @pl.when(pl.program_id(0) == 0)
    def _():
        C = tri_ref.shape[0]
        ri = lax.broadcasted_iota(jnp.int32, (C, C), 0)
        ci = lax.broadcasted_iota(jnp.int32, (C, C), 1)
        tri_ref[...] = (ri >= ci).astype(jnp.float32)
        w1t = _top8_softmax_weights_t(
            jnp.swapaxes(cm2_ref[...], 0, 1), tri_ref[...]
        )
        l1_ref[...] = lax.dot_general(
            w1t, root_ref[...], _CONTRACT0, preferred_element_type=jnp.float32
        )

    wt = _top8_softmax_weights_t(jnp.swapaxes(g_ref[...], 0, 1), tri_ref[...])
    out_ref[...] = lax.dot_general(
        wt, l1_ref[...], _CONTRACT0, preferred_element_type=jnp.float32
    )


@functools.cache
def _make_tc_combine(B, C1, C2, E, blk):
    grid = (B // blk,)
    return pl.pallas_call(
        _tc_body,
        grid=grid,
        in_specs=[
            pl.BlockSpec((blk, C1), lambda i: (i, 0)),
            pl.BlockSpec((C1, C2), lambda i: (0, 0)),
            pl.BlockSpec((C2, E), lambda i: (0, 0)),
        ],
        out_specs=pl.BlockSpec((blk, E), lambda i: (i, 0)),
        out_shape=jax.ShapeDtypeStruct((B, E), jnp.float32),
        scratch_shapes=[
            pltpu.VMEM((C1, E), jnp.float32),
            pltpu.VMEM((C1, C1), jnp.float32),
        ],
    )


def kernel(ids, rootMatrix, connectionMatrix_1, connectionMatrix_2):
    V, C1 = connectionMatrix_1.shape
    C1_, C2 = connectionMatrix_2.shape
    C2_, E = rootMatrix.shape
    (B,) = ids.shape
    ids32 = ids.astype(jnp.int32)
    nch = 2
    ch = B // nch
    outs = []
    for i in range(nch):
        g = _make_sc_gather(V, C1, ch)(
            connectionMatrix_1, ids32[i * ch : (i + 1) * ch]
        )
        outs.append(
            _make_tc_combine(ch, C1, C2, E, 2048)(
                g, connectionMatrix_2, rootMatrix
            )
        )
    return jnp.concatenate(outs, axis=0)


# speedup vs baseline: 1.0505x; 1.0505x over previous
import functools

import jax
import jax.numpy as jnp
from jax import lax
from jax.experimental import pallas as pl
from jax.experimental.pallas import tpu as pltpu
from jax.experimental.pallas import tpu_sc as plsc

TOPK = 8


@functools.cache
def _make_sc_gather(V, D, B):
    info = plsc.get_sparse_core_info()
    NW = info.num_cores * info.num_subcores
    assert B % NW == 0
    b_per_w = B // NW
    CH = min(64, b_per_w)
    assert b_per_w % CH == 0
    n_ch = b_per_w // CH
    mesh = plsc.VectorSubcoreMesh(core_axis_name="c", subcore_axis_name="s")

    @functools.partial(
        pl.kernel,
        mesh=mesh,
        out_type=jax.ShapeDtypeStruct((B, D), jnp.float32),
        scratch_types=[
            pltpu.VMEM((b_per_w,), jnp.int32),
            pltpu.VMEM((2, CH, D), jnp.float32),
            pltpu.SemaphoreType.DMA((2,)),
            pltpu.SemaphoreType.DMA((2,)),
        ],
    )
    def gather_k(table_hbm, idx_hbm, out_hbm, idx_v, rows, gsem, osem):
        wid = lax.axis_index("s") * info.num_cores + lax.axis_index("c")
        base = wid * b_per_w
        pltpu.sync_copy(idx_hbm.at[pl.ds(base, b_per_w)], idx_v)

        def g_copy(c, slot):
            return pltpu.make_async_copy(
                table_hbm.at[idx_v.at[pl.ds(c * CH, CH)]],
                rows.at[slot],
                gsem.at[slot],
            )

        def o_copy(c, slot):
            return pltpu.make_async_copy(
                rows.at[slot],
                out_hbm.at[pl.ds(base + c * CH, CH)],
                osem.at[slot],
            )

        g_copy(0, 0).start()
        for c in range(n_ch):
            slot = c % 2
            if c + 1 < n_ch:
                if c >= 1:
                    o_copy(c - 1, 1 - slot).wait()
                g_copy(c + 1, 1 - slot).start()
            g_copy(c, slot).wait()
            o_copy(c, slot).start()
        if n_ch >= 2:
            o_copy(n_ch - 2, n_ch % 2).wait()
        o_copy(n_ch - 1, (n_ch - 1) % 2).wait()

    return gather_k


_SORT8 = (
    (0, 1), (2, 3), (4, 5), (6, 7),
    (0, 2), (1, 3), (4, 6), (5, 7),
    (1, 2), (5, 6),
    (0, 4), (1, 5), (2, 6), (3, 7),
    (2, 4), (3, 5),
    (1, 2), (3, 4), (5, 6),
)
_BITONIC8 = (
    (0, 4), (1, 5), (2, 6), (3, 7),
    (0, 2), (1, 3), (4, 6), (5, 7),
    (0, 1), (2, 3), (4, 5), (6, 7),
)


def _cmpswap(rows, net):
    for a, b in net:
        hi = jnp.maximum(rows[a], rows[b])
        lo = jnp.minimum(rows[a], rows[b])
        rows[a], rows[b] = hi, lo
    return rows


def _merge_top8(a, b):
    c = [jnp.maximum(a[i], b[7 - i]) for i in range(8)]
    return _cmpswap(c, _BITONIC8)


def _roll0(x, k):
    return jnp.concatenate([x[k:], x[:k]], axis=0)


def _top8_vals(xt):
    C, R = xt.shape
    nvr = C // 8
    rows = [xt[8 * v : 8 * v + 8] for v in range(nvr)]
    lists = []
    for g in range(nvr // 8):
        lists.append(_cmpswap(rows[8 * g : 8 * g + 8], _SORT8))
    while len(lists) > 1:
        lists = [
            _merge_top8(lists[i], lists[i + 1])
            for i in range(0, len(lists), 2)
        ]
    lst = lists[0]
    for d in (4, 2, 1):
        rolled = [_roll0(x, d) for x in lst]
        lst = _cmpswap(
            [jnp.maximum(lst[i], rolled[7 - i]) for i in range(8)], _BITONIC8
        )
    return lst[0], lst[7]


def _top8_softmax_weights_t(xt, tri, ew_ref):
    C, R = xt.shape
    v0, v8 = _top8_vals(xt)
    x3 = xt.reshape(C // 8, 8, R)
    sel = (x3 >= v8[None]).reshape(C, R)
    cnt = jnp.sum(sel.astype(jnp.float32), axis=0, keepdims=True)
    ew_ref[:C, :R] = jnp.where(
        sel, jnp.exp(x3 - v0[None]).reshape(C, R), 0.0
    )

    @pl.when(jnp.max(cnt) > TOPK)
    def _():
        eqm = (x3 == v8[None]).reshape(C, R).astype(jnp.float32)
        need = cnt - TOPK
        r = lax.dot_general(
            tri[:C, :C], eqm, (((1,), (0,)), ((), ())),
            preferred_element_type=jnp.float32,
        )
        drop = jnp.logical_and(eqm > 0.0, r <= need)
        ew_ref[:C, :R] = jnp.where(drop, 0.0, ew_ref[:C, :R])

    e = ew_ref[:C, :R]
    return e / jnp.sum(e, axis=0, keepdims=True)


_CONTRACT0 = (((0,), (0,)), ((), ()))


def _tc_body(g_ref, cm2_ref, root_ref, out_ref, l1_ref, tri_ref, ew_ref):
    E = out_ref.shape[1]

    @pl.when(pl.program_id(0) == 0)
    def _():
        C = tri_ref.shape[0]
        ri = lax.broadcasted_iota(jnp.int32, (C, C), 0)
        ci = lax.broadcasted_iota(jnp.int32, (C, C), 1)
        tri_ref[...] = (ri >= ci).astype(jnp.float32)
        w1t = _top8_softmax_weights_t(
            jnp.swapaxes(cm2_ref[...], 0, 1), tri_ref[...], ew_ref
        )
        l1_ref[...] = lax.dot_general(
            w1t, root_ref[...], _CONTRACT0, preferred_element_type=jnp.float32
        )

    wt = _top8_softmax_weights_t(
        jnp.swapaxes(g_ref[...], 0, 1), tri_ref[...], ew_ref
    )
    out_ref[...] = lax.dot_general(
        wt, l1_ref[...], _CONTRACT0, preferred_element_type=jnp.float32
    )


@functools.cache
def _make_tc_combine(B, C1, C2, E, blk):
    grid = (B // blk,)
    return pl.pallas_call(
        _tc_body,
        grid=grid,
        in_specs=[
            pl.BlockSpec((blk, C1), lambda i: (i, 0)),
            pl.BlockSpec((C1, C2), lambda i: (0, 0)),
            pl.BlockSpec((C2, E), lambda i: (0, 0)),
        ],
        out_specs=pl.BlockSpec((blk, E), lambda i: (i, 0)),
        out_shape=jax.ShapeDtypeStruct((B, E), jnp.float32),
        scratch_shapes=[
            pltpu.VMEM((C1, E), jnp.float32),
            pltpu.VMEM((C1, C1), jnp.float32),
            pltpu.VMEM((C1, blk), jnp.float32),
        ],
    )


def kernel(ids, rootMatrix, connectionMatrix_1, connectionMatrix_2):
    V, C1 = connectionMatrix_1.shape
    C1_, C2 = connectionMatrix_2.shape
    C2_, E = rootMatrix.shape
    (B,) = ids.shape
    ids32 = ids.astype(jnp.int32)
    nch = 2
    ch = B // nch
    outs = []
    for i in range(nch):
        g = _make_sc_gather(V, C1, ch)(
            connectionMatrix_1, ids32[i * ch : (i + 1) * ch]
        )
        outs.append(
            _make_tc_combine(ch, C1, C2, E, 2048)(
                g, connectionMatrix_2, rootMatrix
            )
        )
    return jnp.concatenate(outs, axis=0)


# speedup vs baseline: 1.0582x; 1.0074x over previous
import functools

import jax
import jax.numpy as jnp
from jax import lax
from jax.experimental import pallas as pl
from jax.experimental.pallas import tpu as pltpu
from jax.experimental.pallas import tpu_sc as plsc

TOPK = 8


@functools.cache
def _make_sc_gather(V, D, B):
    info = plsc.get_sparse_core_info()
    NW = info.num_cores * info.num_subcores
    assert B % NW == 0
    b_per_w = B // NW
    CH = min(64, b_per_w)
    assert b_per_w % CH == 0
    n_ch = b_per_w // CH
    mesh = plsc.VectorSubcoreMesh(core_axis_name="c", subcore_axis_name="s")

    @functools.partial(
        pl.kernel,
        mesh=mesh,
        out_type=jax.ShapeDtypeStruct((B, D), jnp.float32),
        scratch_types=[
            pltpu.VMEM((b_per_w,), jnp.int32),
            pltpu.VMEM((2, CH, D), jnp.float32),
            pltpu.SemaphoreType.DMA((2,)),
            pltpu.SemaphoreType.DMA((2,)),
        ],
    )
    def gather_k(table_hbm, idx_hbm, out_hbm, idx_v, rows, gsem, osem):
        wid = lax.axis_index("s") * info.num_cores + lax.axis_index("c")
        base = wid * b_per_w
        pltpu.sync_copy(idx_hbm.at[pl.ds(base, b_per_w)], idx_v)

        def g_copy(c, slot):
            return pltpu.make_async_copy(
                table_hbm.at[idx_v.at[pl.ds(c * CH, CH)]],
                rows.at[slot],
                gsem.at[slot],
            )

        def o_copy(c, slot):
            return pltpu.make_async_copy(
                rows.at[slot],
                out_hbm.at[pl.ds(base + c * CH, CH)],
                osem.at[slot],
            )

        g_copy(0, 0).start()
        for c in range(n_ch):
            slot = c % 2
            if c + 1 < n_ch:
                if c >= 1:
                    o_copy(c - 1, 1 - slot).wait()
                g_copy(c + 1, 1 - slot).start()
            g_copy(c, slot).wait()
            o_copy(c, slot).start()
        if n_ch >= 2:
            o_copy(n_ch - 2, n_ch % 2).wait()
        o_copy(n_ch - 1, (n_ch - 1) % 2).wait()

    return gather_k


_SORT8 = (
    (0, 1), (2, 3), (4, 5), (6, 7),
    (0, 2), (1, 3), (4, 6), (5, 7),
    (1, 2), (5, 6),
    (0, 4), (1, 5), (2, 6), (3, 7),
    (2, 4), (3, 5),
    (1, 2), (3, 4), (5, 6),
)
_BITONIC8 = (
    (0, 4), (1, 5), (2, 6), (3, 7),
    (0, 2), (1, 3), (4, 6), (5, 7),
    (0, 1), (2, 3), (4, 5), (6, 7),
)


def _cmpswap(rows, net):
    for a, b in net:
        hi = jnp.maximum(rows[a], rows[b])
        lo = jnp.minimum(rows[a], rows[b])
        rows[a], rows[b] = hi, lo
    return rows


def _merge_top8(a, b):
    c = [jnp.maximum(a[i], b[7 - i]) for i in range(8)]
    return _cmpswap(c, _BITONIC8)


def _roll0(x, k):
    return jnp.concatenate([x[k:], x[:k]], axis=0)


def _top8_vals(xt):
    C, R = xt.shape
    nvr = C // 8
    rows = [xt[8 * v : 8 * v + 8] for v in range(nvr)]
    lists = []
    for g in range(nvr // 8):
        lists.append(_cmpswap(rows[8 * g : 8 * g + 8], _SORT8))
    while len(lists) > 1:
        lists = [
            _merge_top8(lists[i], lists[i + 1])
            for i in range(0, len(lists), 2)
        ]
    lst = lists[0]
    for d in (4, 2, 1):
        rolled = [_roll0(x, d) for x in lst]
        lst = _cmpswap(
            [jnp.maximum(lst[i], rolled[7 - i]) for i in range(8)], _BITONIC8
        )
    return lst[0], lst[7]


def _top8_softmax_weights_t(xt, tri, ew_ref):
    C, R = xt.shape
    v0, v8 = _top8_vals(xt)
    x3 = xt.reshape(C // 8, 8, R)
    sel = (x3 >= v8[None]).reshape(C, R)
    cnt = jnp.sum(sel.astype(jnp.float32), axis=0, keepdims=True)
    ew_ref[:C, :R] = jnp.where(
        sel, jnp.exp(x3 - v0[None]).reshape(C, R), 0.0
    )

    @pl.when(jnp.max(cnt) > TOPK)
    def _():
        eqm = (x3 == v8[None]).reshape(C, R).astype(jnp.float32)
        need = cnt - TOPK
        r = lax.dot_general(
            tri[:C, :C], eqm, (((1,), (0,)), ((), ())),
            preferred_element_type=jnp.float32,
        )
        drop = jnp.logical_and(eqm > 0.0, r <= need)
        ew_ref[:C, :R] = jnp.where(drop, 0.0, ew_ref[:C, :R])

    e = ew_ref[:C, :R]
    return e / jnp.sum(e, axis=0, keepdims=True)


_CONTRACT0 = (((0,), (0,)), ((), ()))


def _tc_body(g_ref, cm2_ref, root_ref, out_ref, l1_ref, tri_ref, ew_ref):
    E = out_ref.shape[1]

    @pl.when(pl.program_id(0) == 0)
    def _():
        C = tri_ref.shape[0]
        ri = lax.broadcasted_iota(jnp.int32, (C, C), 0)
        ci = lax.broadcasted_iota(jnp.int32, (C, C), 1)
        tri_ref[...] = (ri >= ci).astype(jnp.float32)
        w1t = _top8_softmax_weights_t(
            jnp.swapaxes(cm2_ref[...], 0, 1), tri_ref[...], ew_ref
        )
        l1_ref[...] = lax.dot_general(
            w1t, root_ref[...], _CONTRACT0, preferred_element_type=jnp.float32
        )

    wt = _top8_softmax_weights_t(
        jnp.swapaxes(g_ref[...], 0, 1), tri_ref[...], ew_ref
    )
    out_ref[...] = lax.dot_general(
        wt, l1_ref[...], _CONTRACT0, preferred_element_type=jnp.float32
    )


@functools.cache
def _make_tc_combine(B, C1, C2, E, blk):
    grid = (B // blk,)
    return pl.pallas_call(
        _tc_body,
        grid=grid,
        in_specs=[
            pl.BlockSpec((blk, C1), lambda i: (i, 0)),
            pl.BlockSpec((C1, C2), lambda i: (0, 0)),
            pl.BlockSpec((C2, E), lambda i: (0, 0)),
        ],
        out_specs=pl.BlockSpec((blk, E), lambda i: (i, 0)),
        out_shape=jax.ShapeDtypeStruct((B, E), jnp.float32),
        scratch_shapes=[
            pltpu.VMEM((C1, E), jnp.float32),
            pltpu.VMEM((C1, C1), jnp.float32),
            pltpu.VMEM((C1, blk), jnp.float32),
        ],
    )


def kernel(ids, rootMatrix, connectionMatrix_1, connectionMatrix_2):
    V, C1 = connectionMatrix_1.shape
    C1_, C2 = connectionMatrix_2.shape
    C2_, E = rootMatrix.shape
    (B,) = ids.shape
    ids32 = ids.astype(jnp.int32)
    nch = 2
    ch = B // nch
    outs = []
    for i in range(nch):
        g = _make_sc_gather(V, C1, ch)(
            connectionMatrix_1, ids32[i * ch : (i + 1) * ch]
        )
        outs.append(
            _make_tc_combine(ch, C1, C2, E, 4096)(
                g, connectionMatrix_2, rootMatrix
            )
        )
    return jnp.concatenate(outs, axis=0)
